# Initial kernel scaffold; baseline (speedup 1.0000x reference)
import jax
import jax.numpy as jnp
from jax.experimental import pallas as pl


def kernel(instance, W):
    raise NotImplementedError("write your pallas kernel here")


# speedup vs baseline: 2.3190x; 2.3190x over previous
import jax
import jax.numpy as jnp
from jax import lax
from jax.experimental import pallas as pl
from jax.experimental.pallas import tpu as pltpu
from jax.experimental.pallas import tpu_sc as plsc

NUM_FIELDS = 26
VOCAB = 100000
EMBED_DIM = 32
BATCH = 16384

NUM_CORES = 2
NUM_SUBCORES = 16
NUM_WORKERS = NUM_CORES * NUM_SUBCORES
BPW = BATCH // NUM_WORKERS
CHUNK = 128
NCHUNK = BPW // CHUNK
GROUPS = BPW // 16


def _body(inst_hbm, w_hbm, out_hbm, idx_v, gidx_v, rbuf_v, gsem):
    wid = lax.axis_index("s") * NUM_CORES + lax.axis_index("c")
    base = wid * BPW
    pltpu.sync_copy(inst_hbm.at[pl.ds(base, BPW)], idx_v)

    def field_step(f, carry):
        off = f * VOCAB
        for g in range(GROUPS):
            vals = idx_v[pl.ds(g * 16, 16)] + off
            gidx_v[g // (CHUNK // 16), pl.ds((g % (CHUNK // 16)) * 16, 16)] = vals
        descs = []
        for c in range(NCHUNK):
            descs.append(
                pltpu.async_copy(
                    w_hbm.at[gidx_v.at[c]],
                    rbuf_v.at[pl.ds(c * CHUNK, CHUNK)],
                    gsem,
                )
            )
        for d in descs:
            d.wait()
        pltpu.sync_copy(rbuf_v, out_hbm.at[pl.ds(base, BPW), f])
        return carry

    lax.fori_loop(0, NUM_FIELDS, field_step, 0)


def kernel(instance, W):
    w_flat = W.reshape(NUM_FIELDS * VOCAB, EMBED_DIM)
    idx = instance.astype(jnp.int32)
    mesh = plsc.VectorSubcoreMesh(core_axis_name="c", subcore_axis_name="s")
    out = pl.kernel(
        _body,
        out_type=jax.ShapeDtypeStruct((BATCH, NUM_FIELDS, EMBED_DIM), jnp.float32),
        mesh=mesh,
        scratch_types=[
            pltpu.VMEM((BPW,), jnp.int32),
            pltpu.VMEM((NCHUNK, CHUNK), jnp.int32),
            pltpu.VMEM((BPW, EMBED_DIM), jnp.float32),
            pltpu.SemaphoreType.DMA,
        ],
        compiler_params=pltpu.CompilerParams(use_tc_tiling_on_sc=False),
    )(idx, w_flat)
    return out.reshape(BATCH, NUM_FIELDS * EMBED_DIM)
